# Initial kernel scaffold; baseline (speedup 1.0000x reference)
import jax
import jax.numpy as jnp
from jax.experimental import pallas as pl


def kernel(x, y, t, p, rgb, params):
    raise NotImplementedError("write your pallas kernel here")


# speedup vs baseline: 2.0615x; 2.0615x over previous
import functools

import jax
import jax.numpy as jnp
import numpy as np
from jax import lax
from jax.experimental import pallas as pl
from jax.experimental.pallas import tpu as pltpu

SH, SW, STRIDE, T, K, HID = 215, 320, 8, 8, 10000, 64
HG, WG = SH // STRIDE, SW // STRIDE
EPS = 1e-5
NPIX = SH * SW
NE = 1_000_000
NEP = 1 << 20
NBUF = 16384
TRASH = K
SENT_KEY = np.int32(-0x80000000)
SENT_IDX = np.int32(0x40000000)


def _conv(x, w, b=None, stride=1, pad=0):
    out = lax.conv_general_dilated(x, w, (stride, stride), [(pad, pad), (pad, pad)],
                                   dimension_numbers=('NCHW', 'OIHW', 'NCHW'))
    if b is not None:
        out = out + b[None, :, None, None]
    return out


def _bn(x, g, b):
    return x / jnp.sqrt(1.0 + EPS) * g[None, :, None, None] + b[None, :, None, None]


def _img_encoder(rgb, prm):
    h = jax.nn.relu(_bn(_conv(rgb, prm['stem0_w'], stride=2, pad=1), prm['bn0_g'], prm['bn0_b']))
    h = jax.nn.relu(_bn(_conv(h, prm['stem1_w'], stride=2, pad=1), prm['bn1_g'], prm['bn1_b']))
    h = jax.nn.relu(_bn(_conv(h, prm['stem2_w'], stride=2, pad=1), prm['bn2_g'], prm['bn2_b']))
    return _conv(h, prm['head_w'], prm['head_b'])


def _scorer(voxel, rgb_feat, prm):
    if rgb_feat.shape[-2:] != (HG, WG):
        rgb_feat = jax.image.resize(rgb_feat, (rgb_feat.shape[0], rgb_feat.shape[1], HG, WG),
                                    method='bilinear')
    e = _conv(voxel[None], prm['eproj_w'], prm['eproj_b'])
    r = _conv(rgb_feat, prm['rproj_w'], prm['rproj_b'])
    h = jnp.concatenate([e, r], axis=1)
    h = jax.nn.relu(_bn(_conv(h, prm['fuse0_w'], prm['fuse0_b'], pad=1), prm['fbn_g'], prm['fbn_b']))
    out = _conv(h, prm['fuse1_w'], prm['fuse1_b'])
    return out[0, 0]


def _pixel_score(x, y, sm):
    xn = x.astype(jnp.float32) / SW * 2.0 - 1.0
    yn = y.astype(jnp.float32) / SH * 2.0 - 1.0
    ix = ((xn + 1.0) * WG - 1.0) / 2.0
    iy = ((yn + 1.0) * HG - 1.0) / 2.0
    x0 = jnp.floor(ix); y0 = jnp.floor(iy)
    wx1 = ix - x0; wx0 = 1.0 - wx1
    wy1 = iy - y0; wy0 = 1.0 - wy1
    x0c = jnp.clip(x0, 0, WG - 1).astype(jnp.int32)
    x1c = jnp.clip(x0 + 1, 0, WG - 1).astype(jnp.int32)
    y0c = jnp.clip(y0, 0, HG - 1).astype(jnp.int32)
    y1c = jnp.clip(y0 + 1, 0, HG - 1).astype(jnp.int32)
    return (sm[y0c, x0c] * wy0 * wx0 + sm[y0c, x1c] * wy0 * wx1 +
            sm[y1c, x0c] * wy1 * wx0 + sm[y1c, x1c] * wy1 * wx1)


def _score_table_keys(sm):
    gx = jnp.tile(jnp.arange(SW, dtype=jnp.int32), SH)
    gy = jnp.repeat(jnp.arange(SH, dtype=jnp.int32), SW)
    s = _pixel_score(gx, gy, sm)
    b = lax.bitcast_convert_type(s, jnp.int32)
    kk = jnp.where(b < 0, jnp.bitwise_xor(jnp.bitwise_not(b), jnp.int32(-0x80000000)), b)
    return s, kk


def _theta_body(keys, theta_out, cgt_out):
    kk = keys[...]
    def step(b, c_u):
        trial_u = c_u | (jnp.uint32(1) << jnp.uint32(31 - b))
        trial_i = lax.bitcast_convert_type(trial_u ^ jnp.uint32(0x80000000), jnp.int32)
        cnt = jnp.sum((kk >= trial_i).astype(jnp.int32))
        return jnp.where(cnt >= K, trial_u, c_u)
    c_u = lax.fori_loop(0, 32, step, jnp.uint32(0))
    theta = lax.bitcast_convert_type(c_u ^ jnp.uint32(0x80000000), jnp.int32)
    theta_out[0, 0] = theta
    cgt_out[0, 0] = jnp.sum((kk > theta).astype(jnp.int32))


def _find_theta(keys_evt):
    return pl.pallas_call(
        _theta_body,
        out_shape=(jax.ShapeDtypeStruct((1, 1), jnp.int32),
                   jax.ShapeDtypeStruct((1, 1), jnp.int32)),
        out_specs=(pl.BlockSpec(memory_space=pltpu.SMEM),
                   pl.BlockSpec(memory_space=pltpu.SMEM)),
    )(keys_evt.reshape(8192, 128))


_SCAN_BLKS = 16
_SCAN_R = NEP // _SCAN_BLKS // 128


def _psum_rowmajor(x):
    c = x
    sh = 1
    while sh < 128:
        c = c + jnp.concatenate([jnp.zeros((c.shape[0], sh), jnp.int32), c[:, :-sh]], axis=1)
        sh *= 2
    rows = c[:, -1:]
    r = rows
    sh = 1
    while sh < x.shape[0]:
        r = r + jnp.concatenate([jnp.zeros((sh, 1), jnp.int32), r[:-sh, :]], axis=0)
        sh *= 2
    row_excl = r - rows
    return row_excl + (c - x)


def _pos_body(theta_ref, cgt_ref, keys, pos_out, acc):
    g = pl.program_id(0)

    @pl.when(g == 0)
    def _init():
        acc[0] = 0
        acc[1] = 0

    theta = theta_ref[0, 0]
    cgt = cgt_ref[0, 0]
    m = K - cgt
    kk = keys[...]
    f2 = (kk > theta).astype(jnp.int32)
    f1 = (kk == theta).astype(jnp.int32)
    ps2 = _psum_rowmajor(f2)
    ps1 = _psum_rowmajor(f1)
    base2 = acc[0]
    base1 = acc[1]
    eqr = base1 + ps1
    pos = jnp.where(f2 == 1, base2 + ps2,
                    jnp.where((f1 == 1) & (eqr < m), cgt + eqr, jnp.int32(TRASH)))
    pos_out[...] = pos
    acc[0] = base2 + jnp.sum(f2)
    acc[1] = base1 + jnp.sum(f1)


def _positions(keys_evt, theta, cgt):
    return pl.pallas_call(
        _pos_body,
        grid=(_SCAN_BLKS,),
        in_specs=[
            pl.BlockSpec(memory_space=pltpu.SMEM),
            pl.BlockSpec(memory_space=pltpu.SMEM),
            pl.BlockSpec((_SCAN_R, 128), lambda g: (g, 0)),
        ],
        out_specs=pl.BlockSpec((_SCAN_R, 128), lambda g: (g, 0)),
        out_shape=jax.ShapeDtypeStruct((_SCAN_BLKS * _SCAN_R, 128), jnp.int32),
        scratch_shapes=[pltpu.SMEM((2,), jnp.int32)],
    )(theta, cgt, keys_evt.reshape(_SCAN_BLKS * _SCAN_R, 128))


def _roll1d(a, shift):
    if shift > 0:
        return jnp.concatenate([a[-shift:], a[:-shift]])
    s = -shift
    return jnp.concatenate([a[s:], a[:s]])


def _sort_body(keys, idxs, sc_out, idx_out):
    n = NBUF
    i1 = lax.broadcasted_iota(jnp.int32, (n,), 0)
    kk = jnp.where(i1 >= K, SENT_KEY, keys[...])
    ii = jnp.where(i1 >= K, SENT_IDX, idxs[...])
    for ksz_log in range(1, 15):
        ksz = 1 << ksz_log
        j = ksz >> 1
        while j >= 1:
            lower = (i1 & j) == 0
            pk = jnp.where(lower, _roll1d(kk, -j), _roll1d(kk, j))
            pi = jnp.where(lower, _roll1d(ii, -j), _roll1d(ii, j))
            desc = (i1 & ksz) == 0
            pbetter = (pk > kk) | ((pk == kk) & (pi < ii))
            want_first = desc == lower
            take = want_first == pbetter
            kk = jnp.where(take, pk, kk)
            ii = jnp.where(take, pi, ii)
            j >>= 1
    neg = kk >= 0
    b = jnp.where(neg, kk, jnp.bitwise_not(jnp.bitwise_xor(kk, jnp.int32(-0x80000000))))
    sc_out[...] = lax.bitcast_convert_type(b, jnp.float32)
    idx_out[...] = ii


def _sort_topk(keybuf, idxbuf):
    return pl.pallas_call(
        _sort_body,
        out_shape=(jax.ShapeDtypeStruct((NBUF,), jnp.float32),
                   jax.ShapeDtypeStruct((NBUF,), jnp.int32)),
    )(keybuf, idxbuf)


def _voxelize_xla(x, y, t, p):
    xi = jnp.clip(x.astype(jnp.int32) // STRIDE, 0, WG - 1)
    yi = jnp.clip(y.astype(jnp.int32) // STRIDE, 0, HG - 1)
    ti = jnp.clip((t * T).astype(jnp.int32), 0, T - 1)
    pi = (p > 0).astype(jnp.int32)
    flat = pi * (T * HG * WG) + ti * (HG * WG) + yi * WG + xi
    voxel = jnp.zeros(2 * T * HG * WG, jnp.float32).at[flat].add(1.0)
    return voxel.reshape(2 * T, HG, WG)


def kernel(x, y, t, p, rgb, params):
    voxel = _voxelize_xla(x, y, t, p)
    rgb_f = _img_encoder(rgb, params)
    sm = _scorer(voxel, rgb_f, params)

    _s_tab, k_tab = _score_table_keys(sm)

    pix = y.astype(jnp.int32) * SW + x.astype(jnp.int32)
    keys = k_tab[pix]
    keys_p = jnp.full((NEP,), SENT_KEY, jnp.int32).at[:NE].set(keys)

    theta, cgt = _find_theta(keys_p)
    pos = _positions(keys_p, theta, cgt).reshape(-1)[:NE]

    keybuf = jnp.zeros((NBUF,), jnp.int32).at[pos].set(keys, mode='drop')
    idxbuf = jnp.zeros((NBUF,), jnp.int32).at[pos].set(
        jnp.arange(NE, dtype=jnp.int32), mode='drop')

    top_scores, idx = _sort_topk(keybuf, idxbuf)
    top_scores = top_scores[:K]
    idx = idx[:K]

    return (x[idx], y[idx], t[idx], p[idx], top_scores, sm)


# speedup vs baseline: 9.2847x; 4.5038x over previous
import functools

import jax
import jax.numpy as jnp
import numpy as np
from jax import lax
from jax.experimental import pallas as pl
from jax.experimental.pallas import tpu as pltpu
from jax.experimental.pallas import tpu_sc as plsc

_NC, _NS = 2, 16
_NW = _NC * _NS
_EPT = (1 << 20) // _NW
_CH = 8192
_NSUB = _EPT // _CH
_TPAD = 68864

SH, SW, STRIDE, T, K, HID = 215, 320, 8, 8, 10000, 64
HG, WG = SH // STRIDE, SW // STRIDE
EPS = 1e-5
NPIX = SH * SW
NE = 1_000_000
NEP = 1 << 20
NBUF = 16384
TRASH = K
SENT_KEY = np.int32(-0x80000000)
SENT_IDX = np.int32(0x40000000)


def _conv(x, w, b=None, stride=1, pad=0):
    out = lax.conv_general_dilated(x, w, (stride, stride), [(pad, pad), (pad, pad)],
                                   dimension_numbers=('NCHW', 'OIHW', 'NCHW'))
    if b is not None:
        out = out + b[None, :, None, None]
    return out


def _bn(x, g, b):
    return x / jnp.sqrt(1.0 + EPS) * g[None, :, None, None] + b[None, :, None, None]


def _img_encoder(rgb, prm):
    h = jax.nn.relu(_bn(_conv(rgb, prm['stem0_w'], stride=2, pad=1), prm['bn0_g'], prm['bn0_b']))
    h = jax.nn.relu(_bn(_conv(h, prm['stem1_w'], stride=2, pad=1), prm['bn1_g'], prm['bn1_b']))
    h = jax.nn.relu(_bn(_conv(h, prm['stem2_w'], stride=2, pad=1), prm['bn2_g'], prm['bn2_b']))
    return _conv(h, prm['head_w'], prm['head_b'])


def _scorer(voxel, rgb_feat, prm):
    if rgb_feat.shape[-2:] != (HG, WG):
        rgb_feat = jax.image.resize(rgb_feat, (rgb_feat.shape[0], rgb_feat.shape[1], HG, WG),
                                    method='bilinear')
    e = _conv(voxel[None], prm['eproj_w'], prm['eproj_b'])
    r = _conv(rgb_feat, prm['rproj_w'], prm['rproj_b'])
    h = jnp.concatenate([e, r], axis=1)
    h = jax.nn.relu(_bn(_conv(h, prm['fuse0_w'], prm['fuse0_b'], pad=1), prm['fbn_g'], prm['fbn_b']))
    out = _conv(h, prm['fuse1_w'], prm['fuse1_b'])
    return out[0, 0]


def _pixel_score(x, y, sm):
    xn = x.astype(jnp.float32) / SW * 2.0 - 1.0
    yn = y.astype(jnp.float32) / SH * 2.0 - 1.0
    ix = ((xn + 1.0) * WG - 1.0) / 2.0
    iy = ((yn + 1.0) * HG - 1.0) / 2.0
    x0 = jnp.floor(ix); y0 = jnp.floor(iy)
    wx1 = ix - x0; wx0 = 1.0 - wx1
    wy1 = iy - y0; wy0 = 1.0 - wy1
    x0c = jnp.clip(x0, 0, WG - 1).astype(jnp.int32)
    x1c = jnp.clip(x0 + 1, 0, WG - 1).astype(jnp.int32)
    y0c = jnp.clip(y0, 0, HG - 1).astype(jnp.int32)
    y1c = jnp.clip(y0 + 1, 0, HG - 1).astype(jnp.int32)
    return (sm[y0c, x0c] * wy0 * wx0 + sm[y0c, x1c] * wy0 * wx1 +
            sm[y1c, x0c] * wy1 * wx0 + sm[y1c, x1c] * wy1 * wx1)


def _score_table_keys(sm):
    gx = jnp.tile(jnp.arange(SW, dtype=jnp.int32), SH)
    gy = jnp.repeat(jnp.arange(SH, dtype=jnp.int32), SW)
    s = _pixel_score(gx, gy, sm)
    b = lax.bitcast_convert_type(s, jnp.int32)
    kk = jnp.where(b < 0, jnp.bitwise_xor(jnp.bitwise_not(b), jnp.int32(-0x80000000)), b)
    return s, kk


def _theta_body(keys, theta_out, cgt_out):
    kk = keys[...]
    def step(b, c_u):
        trial_u = c_u | (jnp.uint32(1) << jnp.uint32(31 - b))
        trial_i = lax.bitcast_convert_type(trial_u ^ jnp.uint32(0x80000000), jnp.int32)
        cnt = jnp.sum((kk >= trial_i).astype(jnp.int32))
        return jnp.where(cnt >= K, trial_u, c_u)
    c_u = lax.fori_loop(0, 32, step, jnp.uint32(0))
    theta = lax.bitcast_convert_type(c_u ^ jnp.uint32(0x80000000), jnp.int32)
    theta_out[0, 0] = theta
    cgt_out[0, 0] = jnp.sum((kk > theta).astype(jnp.int32))


def _find_theta(keys_evt):
    return pl.pallas_call(
        _theta_body,
        out_shape=(jax.ShapeDtypeStruct((1, 1), jnp.int32),
                   jax.ShapeDtypeStruct((1, 1), jnp.int32)),
        out_specs=(pl.BlockSpec(memory_space=pltpu.SMEM),
                   pl.BlockSpec(memory_space=pltpu.SMEM)),
    )(keys_evt.reshape(8192, 128))


_SCAN_BLKS = 16
_SCAN_R = NEP // _SCAN_BLKS // 128


def _psum_rowmajor(x):
    c = x
    sh = 1
    while sh < 128:
        c = c + jnp.concatenate([jnp.zeros((c.shape[0], sh), jnp.int32), c[:, :-sh]], axis=1)
        sh *= 2
    rows = c[:, -1:]
    r = rows
    sh = 1
    while sh < x.shape[0]:
        r = r + jnp.concatenate([jnp.zeros((sh, 1), jnp.int32), r[:-sh, :]], axis=0)
        sh *= 2
    row_excl = r - rows
    return row_excl + (c - x)


def _pos_body(theta_ref, cgt_ref, keys, pos_out, acc):
    g = pl.program_id(0)

    @pl.when(g == 0)
    def _init():
        acc[0] = 0
        acc[1] = 0

    theta = theta_ref[0, 0]
    cgt = cgt_ref[0, 0]
    m = K - cgt
    kk = keys[...]
    f2 = (kk > theta).astype(jnp.int32)
    f1 = (kk == theta).astype(jnp.int32)
    ps2 = _psum_rowmajor(f2)
    ps1 = _psum_rowmajor(f1)
    base2 = acc[0]
    base1 = acc[1]
    eqr = base1 + ps1
    pos = jnp.where(f2 == 1, base2 + ps2,
                    jnp.where((f1 == 1) & (eqr < m), cgt + eqr, jnp.int32(TRASH)))
    pos_out[...] = pos
    acc[0] = base2 + jnp.sum(f2)
    acc[1] = base1 + jnp.sum(f1)


def _positions(keys_evt, theta, cgt):
    return pl.pallas_call(
        _pos_body,
        grid=(_SCAN_BLKS,),
        in_specs=[
            pl.BlockSpec(memory_space=pltpu.SMEM),
            pl.BlockSpec(memory_space=pltpu.SMEM),
            pl.BlockSpec((_SCAN_R, 128), lambda g: (g, 0)),
        ],
        out_specs=pl.BlockSpec((_SCAN_R, 128), lambda g: (g, 0)),
        out_shape=jax.ShapeDtypeStruct((_SCAN_BLKS * _SCAN_R, 128), jnp.int32),
        scratch_shapes=[pltpu.SMEM((2,), jnp.int32)],
    )(theta, cgt, keys_evt.reshape(_SCAN_BLKS * _SCAN_R, 128))


def _sc_gather_keys(table_pad, pix_pad):
    mesh = plsc.VectorSubcoreMesh(core_axis_name="c", subcore_axis_name="s", num_cores=_NC)

    @functools.partial(
        pl.kernel, mesh=mesh,
        out_type=jax.ShapeDtypeStruct((NEP,), jnp.int32),
        compiler_params=pltpu.CompilerParams(needs_layout_passes=False),
        scratch_types=[
            pltpu.VMEM((_TPAD,), jnp.int32),
            pltpu.VMEM((_CH,), jnp.int32),
            pltpu.VMEM((_CH,), jnp.int32),
        ],
    )
    def k(table_hbm, pix_hbm, keys_hbm, table_v, pix_v, key_v):
        wid = lax.axis_index("s") * _NC + lax.axis_index("c")
        base = wid * _EPT
        pltpu.sync_copy(table_hbm, table_v)
        for sub in range(_NSUB):
            off = base + sub * _CH
            pltpu.sync_copy(pix_hbm.at[pl.ds(off, _CH)], pix_v)

            def body(j, _):
                pv = pix_v[pl.ds(j * 16, 16)]
                key_v[pl.ds(j * 16, 16)] = plsc.load_gather(table_v, [pv])
                return 0

            lax.fori_loop(0, _CH // 16, body, 0)
            pltpu.sync_copy(key_v, keys_hbm.at[pl.ds(off, _CH)])

    return k(table_pad, pix_pad)


def _sc_scatter_body(keys_hbm, pos_hbm, kout_hbm, iout_hbm,
                     key_v, pos_v, idx_v, sent_v, kbuf_s, ibuf_s):
    cid = lax.axis_index("c")
    sid = lax.axis_index("s")
    wid = sid * _NC + cid
    base = wid * _EPT
    iota16 = lax.broadcasted_iota(jnp.int32, (16,), 0)

    def initb(j, _):
        sent_v[pl.ds(j * 16, 16)] = jnp.full((16,), SENT_KEY, jnp.int32)
        return 0
    lax.fori_loop(0, (NBUF // _NS) // 16, initb, 0)
    pltpu.sync_copy(sent_v, kbuf_s.at[pl.ds(sid * (NBUF // _NS), NBUF // _NS)])
    pltpu.sync_copy(sent_v, ibuf_s.at[pl.ds(sid * (NBUF // _NS), NBUF // _NS)])
    plsc.subcore_barrier()

    for sub in range(_NSUB):
        off = base + sub * _CH
        row = wid * (_EPT // 128) + sub * (_CH // 128)
        pltpu.sync_copy(keys_hbm.at[pl.ds(row, _CH // 128)], key_v)
        pltpu.sync_copy(pos_hbm.at[pl.ds(row, _CH // 128)], pos_v)

        def fill(j, _):
            idx_v[j, pl.ds(0, 16)] = off + j * 128 + iota16
            idx_v[j, pl.ds(16, 16)] = off + j * 128 + 16 + iota16
            idx_v[j, pl.ds(32, 16)] = off + j * 128 + 32 + iota16
            idx_v[j, pl.ds(48, 16)] = off + j * 128 + 48 + iota16
            idx_v[j, pl.ds(64, 16)] = off + j * 128 + 64 + iota16
            idx_v[j, pl.ds(80, 16)] = off + j * 128 + 80 + iota16
            idx_v[j, pl.ds(96, 16)] = off + j * 128 + 96 + iota16
            idx_v[j, pl.ds(112, 16)] = off + j * 128 + 112 + iota16
            return 0
        lax.fori_loop(0, _CH // 128, fill, 0)

        def scat(r, _):
            pltpu.sync_copy(key_v.at[r], kbuf_s.at[pos_v.at[r]])
            pltpu.sync_copy(idx_v.at[r], ibuf_s.at[pos_v.at[r]])
            return 0
        lax.fori_loop(0, _CH // 128, scat, 0)

    plsc.subcore_barrier()

    @pl.when(sid == 0)
    def _out():
        pltpu.sync_copy(kbuf_s, kout_hbm.at[cid])
        pltpu.sync_copy(ibuf_s, iout_hbm.at[cid])


def _sc_scatter(keys_p, pos_p):
    mesh = plsc.VectorSubcoreMesh(core_axis_name="c", subcore_axis_name="s", num_cores=_NC)
    key2d = keys_p.reshape(NEP // 128, 128)
    pos2d = pos_p.reshape(NEP // 128, 128)

    @functools.partial(
        pl.kernel, mesh=mesh,
        out_type=(jax.ShapeDtypeStruct((_NC, NBUF), jnp.int32),
                  jax.ShapeDtypeStruct((_NC, NBUF), jnp.int32)),
        compiler_params=pltpu.CompilerParams(needs_layout_passes=False),
        scratch_types=[
            pltpu.VMEM((_CH // 128, 128), jnp.int32),
            pltpu.VMEM((_CH // 128, 128), jnp.int32),
            pltpu.VMEM((_CH // 128, 128), jnp.int32),
            pltpu.VMEM((NBUF // _NS,), jnp.int32),
            pltpu.VMEM_SHARED((NBUF,), jnp.int32),
            pltpu.VMEM_SHARED((NBUF,), jnp.int32),
        ],
    )
    def k(keys_hbm, pos_hbm, kout_hbm, iout_hbm,
          key_v, pos_v, idx_v, sent_v, kbuf_s, ibuf_s):
        _sc_scatter_body(keys_hbm, pos_hbm, kout_hbm, iout_hbm,
                         key_v, pos_v, idx_v, sent_v, kbuf_s, ibuf_s)

    return k(key2d, pos2d)


def _roll1d(a, shift):
    if shift > 0:
        return jnp.concatenate([a[-shift:], a[:-shift]])
    s = -shift
    return jnp.concatenate([a[s:], a[:s]])


def _sort_body(keys, idxs, sc_out, idx_out):
    n = NBUF
    i1 = lax.broadcasted_iota(jnp.int32, (n,), 0)
    k2 = keys[...]
    i2 = idxs[...]
    use1 = k2[0] == SENT_KEY
    kraw = jnp.where(use1, k2[1], k2[0])
    iraw = jnp.where(use1, i2[1], i2[0])
    kk = jnp.where(i1 >= K, SENT_KEY, kraw)
    ii = jnp.where(i1 >= K, SENT_IDX, iraw)
    for ksz_log in range(1, 15):
        ksz = 1 << ksz_log
        j = ksz >> 1
        while j >= 1:
            lower = (i1 & j) == 0
            pk = jnp.where(lower, _roll1d(kk, -j), _roll1d(kk, j))
            pi = jnp.where(lower, _roll1d(ii, -j), _roll1d(ii, j))
            desc = (i1 & ksz) == 0
            pbetter = (pk > kk) | ((pk == kk) & (pi < ii))
            want_first = desc == lower
            take = want_first == pbetter
            kk = jnp.where(take, pk, kk)
            ii = jnp.where(take, pi, ii)
            j >>= 1
    neg = kk >= 0
    b = jnp.where(neg, kk, jnp.bitwise_not(jnp.bitwise_xor(kk, jnp.int32(-0x80000000))))
    sc_out[...] = lax.bitcast_convert_type(b, jnp.float32)
    idx_out[...] = ii


def _sort_topk(keybuf2, idxbuf2):
    return pl.pallas_call(
        _sort_body,
        out_shape=(jax.ShapeDtypeStruct((NBUF,), jnp.float32),
                   jax.ShapeDtypeStruct((NBUF,), jnp.int32)),
    )(keybuf2, idxbuf2)


def _voxelize_xla(x, y, t, p):
    xi = jnp.clip(x.astype(jnp.int32) // STRIDE, 0, WG - 1)
    yi = jnp.clip(y.astype(jnp.int32) // STRIDE, 0, HG - 1)
    ti = jnp.clip((t * T).astype(jnp.int32), 0, T - 1)
    pi = (p > 0).astype(jnp.int32)
    flat = pi * (T * HG * WG) + ti * (HG * WG) + yi * WG + xi
    voxel = jnp.zeros(2 * T * HG * WG, jnp.float32).at[flat].add(1.0)
    return voxel.reshape(2 * T, HG, WG)


def kernel(x, y, t, p, rgb, params):
    voxel = _voxelize_xla(x, y, t, p)
    rgb_f = _img_encoder(rgb, params)
    sm = _scorer(voxel, rgb_f, params)

    _s_tab, k_tab = _score_table_keys(sm)
    table_pad = jnp.full((_TPAD,), SENT_KEY, jnp.int32).at[:NPIX].set(k_tab)

    pix = y.astype(jnp.int32) * SW + x.astype(jnp.int32)
    pix_pad = jnp.full((NEP,), NPIX, jnp.int32).at[:NE].set(pix)

    keys_p = _sc_gather_keys(table_pad, pix_pad)

    theta, cgt = _find_theta(keys_p)
    pos_p = _positions(keys_p, theta, cgt).reshape(-1)

    keybuf2, idxbuf2 = _sc_scatter(keys_p, pos_p)

    top_scores, idx = _sort_topk(keybuf2, idxbuf2)
    top_scores = top_scores[:K]
    idx = idx[:K]

    return (x[idx], y[idx], t[idx], p[idx], top_scores, sm)


# speedup vs baseline: 12.1949x; 1.3134x over previous
import functools

import jax
import jax.numpy as jnp
import numpy as np
from jax import lax
from jax.experimental import pallas as pl
from jax.experimental.pallas import tpu as pltpu
from jax.experimental.pallas import tpu_sc as plsc

_NC, _NS = 2, 16
_NW = _NC * _NS
_EPT = (1 << 20) // _NW
_CH = 8192
_NSUB = _EPT // _CH
_TPAD = 68864

SH, SW, STRIDE, T, K, HID = 215, 320, 8, 8, 10000, 64
HG, WG = SH // STRIDE, SW // STRIDE
EPS = 1e-5
NPIX = SH * SW
NE = 1_000_000
NEP = 1 << 20
NBUF = 16384
TRASH = K
SENT_KEY = np.int32(-0x80000000)
SENT_IDX = np.int32(0x40000000)


def _conv(x, w, b=None, stride=1, pad=0):
    out = lax.conv_general_dilated(x, w, (stride, stride), [(pad, pad), (pad, pad)],
                                   dimension_numbers=('NCHW', 'OIHW', 'NCHW'))
    if b is not None:
        out = out + b[None, :, None, None]
    return out


def _bn(x, g, b):
    return x / jnp.sqrt(1.0 + EPS) * g[None, :, None, None] + b[None, :, None, None]


def _img_encoder(rgb, prm):
    h = jax.nn.relu(_bn(_conv(rgb, prm['stem0_w'], stride=2, pad=1), prm['bn0_g'], prm['bn0_b']))
    h = jax.nn.relu(_bn(_conv(h, prm['stem1_w'], stride=2, pad=1), prm['bn1_g'], prm['bn1_b']))
    h = jax.nn.relu(_bn(_conv(h, prm['stem2_w'], stride=2, pad=1), prm['bn2_g'], prm['bn2_b']))
    return _conv(h, prm['head_w'], prm['head_b'])


def _scorer(voxel, rgb_feat, prm):
    if rgb_feat.shape[-2:] != (HG, WG):
        rgb_feat = jax.image.resize(rgb_feat, (rgb_feat.shape[0], rgb_feat.shape[1], HG, WG),
                                    method='bilinear')
    e = _conv(voxel[None], prm['eproj_w'], prm['eproj_b'])
    r = _conv(rgb_feat, prm['rproj_w'], prm['rproj_b'])
    h = jnp.concatenate([e, r], axis=1)
    h = jax.nn.relu(_bn(_conv(h, prm['fuse0_w'], prm['fuse0_b'], pad=1), prm['fbn_g'], prm['fbn_b']))
    out = _conv(h, prm['fuse1_w'], prm['fuse1_b'])
    return out[0, 0]


def _pixel_score(x, y, sm):
    xn = x.astype(jnp.float32) / SW * 2.0 - 1.0
    yn = y.astype(jnp.float32) / SH * 2.0 - 1.0
    ix = ((xn + 1.0) * WG - 1.0) / 2.0
    iy = ((yn + 1.0) * HG - 1.0) / 2.0
    x0 = jnp.floor(ix); y0 = jnp.floor(iy)
    wx1 = ix - x0; wx0 = 1.0 - wx1
    wy1 = iy - y0; wy0 = 1.0 - wy1
    x0c = jnp.clip(x0, 0, WG - 1).astype(jnp.int32)
    x1c = jnp.clip(x0 + 1, 0, WG - 1).astype(jnp.int32)
    y0c = jnp.clip(y0, 0, HG - 1).astype(jnp.int32)
    y1c = jnp.clip(y0 + 1, 0, HG - 1).astype(jnp.int32)
    return (sm[y0c, x0c] * wy0 * wx0 + sm[y0c, x1c] * wy0 * wx1 +
            sm[y1c, x0c] * wy1 * wx0 + sm[y1c, x1c] * wy1 * wx1)


def _score_table_keys(sm):
    gx = jnp.tile(jnp.arange(SW, dtype=jnp.int32), SH)
    gy = jnp.repeat(jnp.arange(SH, dtype=jnp.int32), SW)
    s = _pixel_score(gx, gy, sm)
    b = lax.bitcast_convert_type(s, jnp.int32)
    kk = jnp.where(b < 0, jnp.bitwise_xor(jnp.bitwise_not(b), jnp.int32(-0x80000000)), b)
    return s, kk


def _theta_body(keys, theta_out, cgt_out):
    kk = keys[...]
    def step(b, c_u):
        trial_u = c_u | (jnp.uint32(1) << jnp.uint32(31 - b))
        trial_i = lax.bitcast_convert_type(trial_u ^ jnp.uint32(0x80000000), jnp.int32)
        cnt = jnp.sum((kk >= trial_i).astype(jnp.int32))
        return jnp.where(cnt >= K, trial_u, c_u)
    c_u = lax.fori_loop(0, 32, step, jnp.uint32(0))
    theta = lax.bitcast_convert_type(c_u ^ jnp.uint32(0x80000000), jnp.int32)
    theta_out[0, 0] = theta
    cgt_out[0, 0] = jnp.sum((kk > theta).astype(jnp.int32))


def _find_theta(keys_evt):
    return pl.pallas_call(
        _theta_body,
        out_shape=(jax.ShapeDtypeStruct((1, 1), jnp.int32),
                   jax.ShapeDtypeStruct((1, 1), jnp.int32)),
        out_specs=(pl.BlockSpec(memory_space=pltpu.SMEM),
                   pl.BlockSpec(memory_space=pltpu.SMEM)),
    )(keys_evt.reshape(8192, 128))


_SCAN_BLKS = 16
_SCAN_R = NEP // _SCAN_BLKS // 128


def _psum_rowmajor(x):
    c = x
    sh = 1
    while sh < 128:
        c = c + jnp.concatenate([jnp.zeros((c.shape[0], sh), jnp.int32), c[:, :-sh]], axis=1)
        sh *= 2
    rows = c[:, -1:]
    r = rows
    sh = 1
    while sh < x.shape[0]:
        r = r + jnp.concatenate([jnp.zeros((sh, 1), jnp.int32), r[:-sh, :]], axis=0)
        sh *= 2
    row_excl = r - rows
    return row_excl + (c - x)


def _pos_body(theta_ref, cgt_ref, keys, pos_out, acc):
    g = pl.program_id(0)

    @pl.when(g == 0)
    def _init():
        acc[0] = 0
        acc[1] = 0

    theta = theta_ref[0, 0]
    cgt = cgt_ref[0, 0]
    m = K - cgt
    kk = keys[...]
    f2 = (kk > theta).astype(jnp.int32)
    f1 = (kk == theta).astype(jnp.int32)
    ps2 = _psum_rowmajor(f2)
    ps1 = _psum_rowmajor(f1)
    base2 = acc[0]
    base1 = acc[1]
    eqr = base1 + ps1
    pos = jnp.where(f2 == 1, base2 + ps2,
                    jnp.where((f1 == 1) & (eqr < m), cgt + eqr, jnp.int32(TRASH)))
    pos_out[...] = pos
    acc[0] = base2 + jnp.sum(f2)
    acc[1] = base1 + jnp.sum(f1)


def _positions(keys_evt, theta, cgt):
    return pl.pallas_call(
        _pos_body,
        grid=(_SCAN_BLKS,),
        in_specs=[
            pl.BlockSpec(memory_space=pltpu.SMEM),
            pl.BlockSpec(memory_space=pltpu.SMEM),
            pl.BlockSpec((_SCAN_R, 128), lambda g: (g, 0)),
        ],
        out_specs=pl.BlockSpec((_SCAN_R, 128), lambda g: (g, 0)),
        out_shape=jax.ShapeDtypeStruct((_SCAN_BLKS * _SCAN_R, 128), jnp.int32),
        scratch_shapes=[pltpu.SMEM((2,), jnp.int32)],
    )(theta, cgt, keys_evt.reshape(_SCAN_BLKS * _SCAN_R, 128))


_NVOX = 2 * T * HG * WG
_VOXPAD = 16896


def _sc_voxelize(xp, yp, tp, pp):
    mesh = plsc.VectorSubcoreMesh(core_axis_name="c", subcore_axis_name="s", num_cores=_NC)

    @functools.partial(
        pl.kernel, mesh=mesh,
        out_type=jax.ShapeDtypeStruct((_NC, _VOXPAD), jnp.float32),
        compiler_params=pltpu.CompilerParams(needs_layout_passes=False),
        scratch_types=[
            pltpu.VMEM((_CH,), jnp.int32),
            pltpu.VMEM((_CH,), jnp.int32),
            pltpu.VMEM((_CH,), jnp.float32),
            pltpu.VMEM((_CH,), jnp.int32),
            pltpu.VMEM((_CH // 128, 128), jnp.int32),
            pltpu.VMEM((128,), jnp.float32),
            pltpu.VMEM((_VOXPAD // _NS,), jnp.float32),
            pltpu.VMEM_SHARED((_VOXPAD,), jnp.float32),
        ],
    )
    def k(x_hbm, y_hbm, t_hbm, p_hbm, vout_hbm,
          x_v, y_v, t_v, p_v, flat_v, ones_v, zero_v, vox_s):
        cid = lax.axis_index("c")
        sid = lax.axis_index("s")
        wid = sid * _NC + cid
        base = wid * _EPT
        iota16 = lax.broadcasted_iota(jnp.int32, (16,), 0)

        def zinit(j, _):
            zero_v[pl.ds(j * 16, 16)] = jnp.zeros((16,), jnp.float32)
            return 0
        lax.fori_loop(0, (_VOXPAD // _NS) // 16, zinit, 0)
        for c in range(8):
            ones_v[pl.ds(c * 16, 16)] = jnp.ones((16,), jnp.float32)
        pltpu.sync_copy(zero_v, vox_s.at[pl.ds(sid * (_VOXPAD // _NS), _VOXPAD // _NS)])
        plsc.subcore_barrier()

        for sub in range(_NSUB):
            off = base + sub * _CH
            pltpu.sync_copy(x_hbm.at[pl.ds(off, _CH)], x_v)
            pltpu.sync_copy(y_hbm.at[pl.ds(off, _CH)], y_v)
            pltpu.sync_copy(t_hbm.at[pl.ds(off, _CH)], t_v)
            pltpu.sync_copy(p_hbm.at[pl.ds(off, _CH)], p_v)

            def fill(r, _):
                for c in range(8):
                    sl = pl.ds(r * 128 + c * 16, 16)
                    xi = x_v[sl] >> 3
                    yi = jnp.minimum(y_v[sl] >> 3, HG - 1)
                    ti = jnp.clip((t_v[sl] * float(T)).astype(jnp.int32), 0, T - 1)
                    pib = jnp.where(p_v[sl] > 0, jnp.int32(T * HG * WG), jnp.int32(0))
                    flat = pib + ti * (HG * WG) + yi * WG + xi
                    gid = off + r * 128 + c * 16 + iota16
                    flat_v[r, pl.ds(c * 16, 16)] = jnp.where(gid < NE, flat, jnp.int32(_NVOX))
                return 0
            lax.fori_loop(0, _CH // 128, fill, 0)

            def scat(r, _):
                pltpu.sync_copy(ones_v, vox_s.at[flat_v.at[r]], add=True)
                return 0
            lax.fori_loop(0, _CH // 128, scat, 0)

        plsc.subcore_barrier()

        @pl.when(sid == 0)
        def _out():
            pltpu.sync_copy(vox_s, vout_hbm.at[cid])

    v2 = k(xp, yp, tp, pp)
    return (v2[0, :_NVOX] + v2[1, :_NVOX]).reshape(2 * T, HG, WG)


_GPAD = 32768
_GPT = _GPAD // _NW


def _sc_out_gather(idx_pad, x, y, t, p):
    mesh = plsc.VectorSubcoreMesh(core_axis_name="c", subcore_axis_name="s", num_cores=_NC)

    @functools.partial(
        pl.kernel, mesh=mesh,
        out_type=(jax.ShapeDtypeStruct((_GPAD // 128, 128), jnp.int32),
                  jax.ShapeDtypeStruct((_GPAD // 128, 128), jnp.int32),
                  jax.ShapeDtypeStruct((_GPAD // 128, 128), jnp.float32),
                  jax.ShapeDtypeStruct((_GPAD // 128, 128), jnp.int32)),
        compiler_params=pltpu.CompilerParams(needs_layout_passes=False),
        scratch_types=[
            pltpu.VMEM((_GPT // 128, 128), jnp.int32),
            pltpu.VMEM((_GPT // 128, 128), jnp.int32),
            pltpu.VMEM((_GPT // 128, 128), jnp.int32),
            pltpu.VMEM((_GPT // 128, 128), jnp.float32),
            pltpu.VMEM((_GPT // 128, 128), jnp.int32),
        ],
    )
    def k(idx_hbm, x_hbm, y_hbm, t_hbm, p_hbm,
          xo_hbm, yo_hbm, to_hbm, po_hbm,
          idx_v, xo_v, yo_v, to_v, po_v):
        wid = lax.axis_index("s") * _NC + lax.axis_index("c")
        base = wid * _GPT
        row = pl.multiple_of(base // 128, 8)
        pltpu.sync_copy(idx_hbm.at[pl.ds(row, _GPT // 128)], idx_v)
        for r in range(_GPT // 128):
            pltpu.sync_copy(x_hbm.at[idx_v.at[r]], xo_v.at[r])
            pltpu.sync_copy(y_hbm.at[idx_v.at[r]], yo_v.at[r])
            pltpu.sync_copy(t_hbm.at[idx_v.at[r]], to_v.at[r])
            pltpu.sync_copy(p_hbm.at[idx_v.at[r]], po_v.at[r])
        pltpu.sync_copy(xo_v, xo_hbm.at[pl.ds(row, _GPT // 128)])
        pltpu.sync_copy(yo_v, yo_hbm.at[pl.ds(row, _GPT // 128)])
        pltpu.sync_copy(to_v, to_hbm.at[pl.ds(row, _GPT // 128)])
        pltpu.sync_copy(po_v, po_hbm.at[pl.ds(row, _GPT // 128)])

    return k(idx_pad.reshape(_GPAD // 128, 128), x, y, t, p)


def _sc_gather_keys(table_pad, pix_pad):
    mesh = plsc.VectorSubcoreMesh(core_axis_name="c", subcore_axis_name="s", num_cores=_NC)

    @functools.partial(
        pl.kernel, mesh=mesh,
        out_type=jax.ShapeDtypeStruct((NEP,), jnp.int32),
        compiler_params=pltpu.CompilerParams(needs_layout_passes=False),
        scratch_types=[
            pltpu.VMEM((_TPAD,), jnp.int32),
            pltpu.VMEM((_CH,), jnp.int32),
            pltpu.VMEM((_CH,), jnp.int32),
        ],
    )
    def k(table_hbm, pix_hbm, keys_hbm, table_v, pix_v, key_v):
        wid = lax.axis_index("s") * _NC + lax.axis_index("c")
        base = wid * _EPT
        pltpu.sync_copy(table_hbm, table_v)
        for sub in range(_NSUB):
            off = base + sub * _CH
            pltpu.sync_copy(pix_hbm.at[pl.ds(off, _CH)], pix_v)

            def body(j, _):
                pv = pix_v[pl.ds(j * 16, 16)]
                key_v[pl.ds(j * 16, 16)] = plsc.load_gather(table_v, [pv])
                return 0

            lax.fori_loop(0, _CH // 16, body, 0)
            pltpu.sync_copy(key_v, keys_hbm.at[pl.ds(off, _CH)])

    return k(table_pad, pix_pad)


def _sc_scatter_body(keys_hbm, pos_hbm, kout_hbm, iout_hbm,
                     key_v, pos_v, idx_v, sent_v, kbuf_s, ibuf_s):
    cid = lax.axis_index("c")
    sid = lax.axis_index("s")
    wid = sid * _NC + cid
    base = wid * _EPT
    iota16 = lax.broadcasted_iota(jnp.int32, (16,), 0)

    def initb(j, _):
        sent_v[pl.ds(j * 16, 16)] = jnp.full((16,), SENT_KEY, jnp.int32)
        return 0
    lax.fori_loop(0, (NBUF // _NS) // 16, initb, 0)
    pltpu.sync_copy(sent_v, kbuf_s.at[pl.ds(sid * (NBUF // _NS), NBUF // _NS)])
    pltpu.sync_copy(sent_v, ibuf_s.at[pl.ds(sid * (NBUF // _NS), NBUF // _NS)])
    plsc.subcore_barrier()

    for sub in range(_NSUB):
        off = base + sub * _CH
        row = wid * (_EPT // 128) + sub * (_CH // 128)
        pltpu.sync_copy(keys_hbm.at[pl.ds(row, _CH // 128)], key_v)
        pltpu.sync_copy(pos_hbm.at[pl.ds(row, _CH // 128)], pos_v)

        def fill(j, _):
            idx_v[j, pl.ds(0, 16)] = off + j * 128 + iota16
            idx_v[j, pl.ds(16, 16)] = off + j * 128 + 16 + iota16
            idx_v[j, pl.ds(32, 16)] = off + j * 128 + 32 + iota16
            idx_v[j, pl.ds(48, 16)] = off + j * 128 + 48 + iota16
            idx_v[j, pl.ds(64, 16)] = off + j * 128 + 64 + iota16
            idx_v[j, pl.ds(80, 16)] = off + j * 128 + 80 + iota16
            idx_v[j, pl.ds(96, 16)] = off + j * 128 + 96 + iota16
            idx_v[j, pl.ds(112, 16)] = off + j * 128 + 112 + iota16
            return 0
        lax.fori_loop(0, _CH // 128, fill, 0)

        def scat(r, _):
            pltpu.sync_copy(key_v.at[r], kbuf_s.at[pos_v.at[r]])
            pltpu.sync_copy(idx_v.at[r], ibuf_s.at[pos_v.at[r]])
            return 0
        lax.fori_loop(0, _CH // 128, scat, 0)

    plsc.subcore_barrier()

    @pl.when(sid == 0)
    def _out():
        pltpu.sync_copy(kbuf_s, kout_hbm.at[cid])
        pltpu.sync_copy(ibuf_s, iout_hbm.at[cid])


def _sc_scatter(keys_p, pos_p):
    mesh = plsc.VectorSubcoreMesh(core_axis_name="c", subcore_axis_name="s", num_cores=_NC)
    key2d = keys_p.reshape(NEP // 128, 128)
    pos2d = pos_p.reshape(NEP // 128, 128)

    @functools.partial(
        pl.kernel, mesh=mesh,
        out_type=(jax.ShapeDtypeStruct((_NC, NBUF), jnp.int32),
                  jax.ShapeDtypeStruct((_NC, NBUF), jnp.int32)),
        compiler_params=pltpu.CompilerParams(needs_layout_passes=False),
        scratch_types=[
            pltpu.VMEM((_CH // 128, 128), jnp.int32),
            pltpu.VMEM((_CH // 128, 128), jnp.int32),
            pltpu.VMEM((_CH // 128, 128), jnp.int32),
            pltpu.VMEM((NBUF // _NS,), jnp.int32),
            pltpu.VMEM_SHARED((NBUF,), jnp.int32),
            pltpu.VMEM_SHARED((NBUF,), jnp.int32),
        ],
    )
    def k(keys_hbm, pos_hbm, kout_hbm, iout_hbm,
          key_v, pos_v, idx_v, sent_v, kbuf_s, ibuf_s):
        _sc_scatter_body(keys_hbm, pos_hbm, kout_hbm, iout_hbm,
                         key_v, pos_v, idx_v, sent_v, kbuf_s, ibuf_s)

    return k(key2d, pos2d)


def _roll1d(a, shift):
    if shift > 0:
        return jnp.concatenate([a[-shift:], a[:-shift]])
    s = -shift
    return jnp.concatenate([a[s:], a[:s]])


def _sort_body(keys, idxs, sc_out, idx_out):
    n = NBUF
    i1 = lax.broadcasted_iota(jnp.int32, (n,), 0)
    k2 = keys[...]
    i2 = idxs[...]
    use1 = k2[0] == SENT_KEY
    kraw = jnp.where(use1, k2[1], k2[0])
    iraw = jnp.where(use1, i2[1], i2[0])
    kk = jnp.where(i1 >= K, SENT_KEY, kraw)
    ii = jnp.where(i1 >= K, SENT_IDX, iraw)
    for ksz_log in range(1, 15):
        ksz = 1 << ksz_log
        j = ksz >> 1
        while j >= 1:
            lower = (i1 & j) == 0
            pk = jnp.where(lower, _roll1d(kk, -j), _roll1d(kk, j))
            pi = jnp.where(lower, _roll1d(ii, -j), _roll1d(ii, j))
            desc = (i1 & ksz) == 0
            pbetter = (pk > kk) | ((pk == kk) & (pi < ii))
            want_first = desc == lower
            take = want_first == pbetter
            kk = jnp.where(take, pk, kk)
            ii = jnp.where(take, pi, ii)
            j >>= 1
    neg = kk >= 0
    b = jnp.where(neg, kk, jnp.bitwise_not(jnp.bitwise_xor(kk, jnp.int32(-0x80000000))))
    sc_out[...] = lax.bitcast_convert_type(b, jnp.float32)
    idx_out[...] = ii


def _sort_topk(keybuf2, idxbuf2):
    return pl.pallas_call(
        _sort_body,
        out_shape=(jax.ShapeDtypeStruct((NBUF,), jnp.float32),
                   jax.ShapeDtypeStruct((NBUF,), jnp.int32)),
    )(keybuf2, idxbuf2)


def kernel(x, y, t, p, rgb, params):
    x = x.astype(jnp.int32)
    y = y.astype(jnp.int32)
    p = p.astype(jnp.int32)
    zpad = jnp.zeros((NEP - NE,), jnp.int32)
    xp = jnp.concatenate([x, zpad])
    yp = jnp.concatenate([y, zpad])
    tp = jnp.concatenate([t, zpad.astype(jnp.float32)])
    pp = jnp.concatenate([p, zpad])

    voxel = _sc_voxelize(xp, yp, tp, pp)
    rgb_f = _img_encoder(rgb, params)
    sm = _scorer(voxel, rgb_f, params)

    _s_tab, k_tab = _score_table_keys(sm)
    table_pad = jnp.full((_TPAD,), SENT_KEY, jnp.int32).at[:NPIX].set(k_tab)

    pix = y * SW + x
    pix_pad = jnp.full((NEP,), NPIX, jnp.int32).at[:NE].set(pix)

    keys_p = _sc_gather_keys(table_pad, pix_pad)

    theta, cgt = _find_theta(keys_p)
    pos_p = _positions(keys_p, theta, cgt).reshape(-1)

    keybuf2, idxbuf2 = _sc_scatter(keys_p, pos_p)

    top_scores, idx = _sort_topk(keybuf2, idxbuf2)
    top_scores = top_scores[:K]

    idx_pad = jnp.minimum(jnp.concatenate([idx, jnp.zeros((_GPAD - NBUF,), jnp.int32)]),
                          NE - 1)
    xo, yo, to, po = _sc_out_gather(idx_pad, x, y, t, p)
    xo = xo.reshape(-1)[:K]
    yo = yo.reshape(-1)[:K]
    to = to.reshape(-1)[:K]
    po = po.reshape(-1)[:K]

    return (xo, yo, to, po, top_scores, sm)


# speedup vs baseline: 12.2200x; 1.0021x over previous
import functools

import jax
import jax.numpy as jnp
import numpy as np
from jax import lax
from jax.experimental import pallas as pl
from jax.experimental.pallas import tpu as pltpu
from jax.experimental.pallas import tpu_sc as plsc

_NC, _NS = 2, 16
_NW = _NC * _NS
_EPT = (1 << 20) // _NW
_CH = 8192
_NSUB = _EPT // _CH
_TPAD = 68864

SH, SW, STRIDE, T, K, HID = 215, 320, 8, 8, 10000, 64
HG, WG = SH // STRIDE, SW // STRIDE
EPS = 1e-5
NPIX = SH * SW
NE = 1_000_000
NEP = 1 << 20
NBUF = 16384
TRASH = K
SENT_KEY = np.int32(-0x80000000)
SENT_IDX = np.int32(0x40000000)


def _conv(x, w, b=None, stride=1, pad=0):
    out = lax.conv_general_dilated(x, w, (stride, stride), [(pad, pad), (pad, pad)],
                                   dimension_numbers=('NCHW', 'OIHW', 'NCHW'))
    if b is not None:
        out = out + b[None, :, None, None]
    return out


def _bn(x, g, b):
    return x / jnp.sqrt(1.0 + EPS) * g[None, :, None, None] + b[None, :, None, None]


def _img_encoder(rgb, prm):
    h = jax.nn.relu(_bn(_conv(rgb, prm['stem0_w'], stride=2, pad=1), prm['bn0_g'], prm['bn0_b']))
    h = jax.nn.relu(_bn(_conv(h, prm['stem1_w'], stride=2, pad=1), prm['bn1_g'], prm['bn1_b']))
    h = jax.nn.relu(_bn(_conv(h, prm['stem2_w'], stride=2, pad=1), prm['bn2_g'], prm['bn2_b']))
    return _conv(h, prm['head_w'], prm['head_b'])


def _scorer(voxel, rgb_feat, prm):
    if rgb_feat.shape[-2:] != (HG, WG):
        rgb_feat = jax.image.resize(rgb_feat, (rgb_feat.shape[0], rgb_feat.shape[1], HG, WG),
                                    method='bilinear')
    e = _conv(voxel[None], prm['eproj_w'], prm['eproj_b'])
    r = _conv(rgb_feat, prm['rproj_w'], prm['rproj_b'])
    h = jnp.concatenate([e, r], axis=1)
    h = jax.nn.relu(_bn(_conv(h, prm['fuse0_w'], prm['fuse0_b'], pad=1), prm['fbn_g'], prm['fbn_b']))
    out = _conv(h, prm['fuse1_w'], prm['fuse1_b'])
    return out[0, 0]


def _pixel_score(x, y, sm):
    xn = x.astype(jnp.float32) / SW * 2.0 - 1.0
    yn = y.astype(jnp.float32) / SH * 2.0 - 1.0
    ix = ((xn + 1.0) * WG - 1.0) / 2.0
    iy = ((yn + 1.0) * HG - 1.0) / 2.0
    x0 = jnp.floor(ix); y0 = jnp.floor(iy)
    wx1 = ix - x0; wx0 = 1.0 - wx1
    wy1 = iy - y0; wy0 = 1.0 - wy1
    x0c = jnp.clip(x0, 0, WG - 1).astype(jnp.int32)
    x1c = jnp.clip(x0 + 1, 0, WG - 1).astype(jnp.int32)
    y0c = jnp.clip(y0, 0, HG - 1).astype(jnp.int32)
    y1c = jnp.clip(y0 + 1, 0, HG - 1).astype(jnp.int32)
    return (sm[y0c, x0c] * wy0 * wx0 + sm[y0c, x1c] * wy0 * wx1 +
            sm[y1c, x0c] * wy1 * wx0 + sm[y1c, x1c] * wy1 * wx1)


def _score_table_keys(sm):
    gx = jnp.tile(jnp.arange(SW, dtype=jnp.int32), SH)
    gy = jnp.repeat(jnp.arange(SH, dtype=jnp.int32), SW)
    s = _pixel_score(gx, gy, sm)
    b = lax.bitcast_convert_type(s, jnp.int32)
    kk = jnp.where(b < 0, jnp.bitwise_xor(jnp.bitwise_not(b), jnp.int32(-0x80000000)), b)
    return s, kk


def _theta_body(keys, theta_out, cgt_out):
    kk = keys[...]
    def step(b, c_u):
        trial_u = c_u | (jnp.uint32(1) << jnp.uint32(31 - b))
        trial_i = lax.bitcast_convert_type(trial_u ^ jnp.uint32(0x80000000), jnp.int32)
        cnt = jnp.sum((kk >= trial_i).astype(jnp.int32))
        return jnp.where(cnt >= K, trial_u, c_u)
    c_u = lax.fori_loop(0, 32, step, jnp.uint32(0))
    theta = lax.bitcast_convert_type(c_u ^ jnp.uint32(0x80000000), jnp.int32)
    theta_out[0, 0] = theta
    cgt_out[0, 0] = jnp.sum((kk > theta).astype(jnp.int32))


def _find_theta(keys_evt):
    return pl.pallas_call(
        _theta_body,
        out_shape=(jax.ShapeDtypeStruct((1, 1), jnp.int32),
                   jax.ShapeDtypeStruct((1, 1), jnp.int32)),
        out_specs=(pl.BlockSpec(memory_space=pltpu.SMEM),
                   pl.BlockSpec(memory_space=pltpu.SMEM)),
    )(keys_evt.reshape(8192, 128))


_SCAN_BLKS = 16
_SCAN_R = NEP // _SCAN_BLKS // 128


def _psum_rowmajor(x):
    c = x
    sh = 1
    while sh < 128:
        c = c + jnp.concatenate([jnp.zeros((c.shape[0], sh), jnp.int32), c[:, :-sh]], axis=1)
        sh *= 2
    rows = c[:, -1:]
    r = rows
    sh = 1
    while sh < x.shape[0]:
        r = r + jnp.concatenate([jnp.zeros((sh, 1), jnp.int32), r[:-sh, :]], axis=0)
        sh *= 2
    row_excl = r - rows
    return row_excl + (c - x)


def _pos_body(theta_ref, cgt_ref, keys, pos_out, acc):
    g = pl.program_id(0)

    @pl.when(g == 0)
    def _init():
        acc[0] = 0
        acc[1] = 0

    theta = theta_ref[0, 0]
    cgt = cgt_ref[0, 0]
    m = K - cgt
    kk = keys[...]
    f2 = (kk > theta).astype(jnp.int32)
    f1 = (kk == theta).astype(jnp.int32)
    ps2 = _psum_rowmajor(f2)
    ps1 = _psum_rowmajor(f1)
    base2 = acc[0]
    base1 = acc[1]
    eqr = base1 + ps1
    pos = jnp.where(f2 == 1, base2 + ps2,
                    jnp.where((f1 == 1) & (eqr < m), cgt + eqr, jnp.int32(TRASH)))
    pos_out[...] = pos
    acc[0] = base2 + jnp.sum(f2)
    acc[1] = base1 + jnp.sum(f1)


def _positions(keys_evt, theta, cgt):
    return pl.pallas_call(
        _pos_body,
        grid=(_SCAN_BLKS,),
        in_specs=[
            pl.BlockSpec(memory_space=pltpu.SMEM),
            pl.BlockSpec(memory_space=pltpu.SMEM),
            pl.BlockSpec((_SCAN_R, 128), lambda g: (g, 0)),
        ],
        out_specs=pl.BlockSpec((_SCAN_R, 128), lambda g: (g, 0)),
        out_shape=jax.ShapeDtypeStruct((_SCAN_BLKS * _SCAN_R, 128), jnp.int32),
        scratch_shapes=[pltpu.SMEM((2,), jnp.int32)],
    )(theta, cgt, keys_evt.reshape(_SCAN_BLKS * _SCAN_R, 128))


_NVOX = 2 * T * HG * WG
_VOXPAD = 16896


def _sc_voxelize(xp, yp, tp, pp):
    mesh = plsc.VectorSubcoreMesh(core_axis_name="c", subcore_axis_name="s", num_cores=_NC)

    @functools.partial(
        pl.kernel, mesh=mesh,
        out_type=jax.ShapeDtypeStruct((_NC, _VOXPAD), jnp.float32),
        compiler_params=pltpu.CompilerParams(needs_layout_passes=False),
        scratch_types=[
            pltpu.VMEM((_CH,), jnp.int32),
            pltpu.VMEM((_CH,), jnp.int32),
            pltpu.VMEM((_CH,), jnp.float32),
            pltpu.VMEM((_CH,), jnp.int32),
            pltpu.VMEM((_CH // 128, 128), jnp.int32),
            pltpu.VMEM((128,), jnp.float32),
            pltpu.VMEM((_VOXPAD // _NS,), jnp.float32),
            pltpu.VMEM_SHARED((_VOXPAD,), jnp.float32),
            pltpu.SemaphoreType.DMA,
        ],
    )
    def k(x_hbm, y_hbm, t_hbm, p_hbm, vout_hbm,
          x_v, y_v, t_v, p_v, flat_v, ones_v, zero_v, vox_s, sem):
        cid = lax.axis_index("c")
        sid = lax.axis_index("s")
        wid = sid * _NC + cid
        base = wid * _EPT
        iota16 = lax.broadcasted_iota(jnp.int32, (16,), 0)

        def zinit(j, _):
            zero_v[pl.ds(j * 16, 16)] = jnp.zeros((16,), jnp.float32)
            return 0
        lax.fori_loop(0, (_VOXPAD // _NS) // 16, zinit, 0)
        for c in range(8):
            ones_v[pl.ds(c * 16, 16)] = jnp.ones((16,), jnp.float32)
        pltpu.sync_copy(zero_v, vox_s.at[pl.ds(sid * (_VOXPAD // _NS), _VOXPAD // _NS)])
        plsc.subcore_barrier()

        for sub in range(_NSUB):
            off = base + sub * _CH
            pltpu.sync_copy(x_hbm.at[pl.ds(off, _CH)], x_v)
            pltpu.sync_copy(y_hbm.at[pl.ds(off, _CH)], y_v)
            pltpu.sync_copy(t_hbm.at[pl.ds(off, _CH)], t_v)
            pltpu.sync_copy(p_hbm.at[pl.ds(off, _CH)], p_v)

            def fill(r, _):
                for c in range(8):
                    sl = pl.ds(r * 128 + c * 16, 16)
                    xi = x_v[sl] >> 3
                    yi = jnp.minimum(y_v[sl] >> 3, HG - 1)
                    ti = jnp.clip((t_v[sl] * float(T)).astype(jnp.int32), 0, T - 1)
                    pib = jnp.where(p_v[sl] > 0, jnp.int32(T * HG * WG), jnp.int32(0))
                    flat = pib + ti * (HG * WG) + yi * WG + xi
                    gid = off + r * 128 + c * 16 + iota16
                    flat_v[r, pl.ds(c * 16, 16)] = jnp.where(gid < NE, flat, jnp.int32(_NVOX))
                return 0
            lax.fori_loop(0, _CH // 128, fill, 0)

            handles = []
            for r in range(_CH // 128):
                handles.append(pltpu.async_copy(ones_v, vox_s.at[flat_v.at[r]],
                                                sem, add=True))
            for h in handles:
                h.wait()

        plsc.subcore_barrier()

        @pl.when(sid == 0)
        def _out():
            pltpu.sync_copy(vox_s, vout_hbm.at[cid])

    v2 = k(xp, yp, tp, pp)
    return (v2[0, :_NVOX] + v2[1, :_NVOX]).reshape(2 * T, HG, WG)


_GPAD = 32768
_GPT = _GPAD // _NW


def _sc_out_gather(idx_pad, x, y, t, p):
    mesh = plsc.VectorSubcoreMesh(core_axis_name="c", subcore_axis_name="s", num_cores=_NC)

    @functools.partial(
        pl.kernel, mesh=mesh,
        out_type=(jax.ShapeDtypeStruct((_GPAD // 128, 128), jnp.int32),
                  jax.ShapeDtypeStruct((_GPAD // 128, 128), jnp.int32),
                  jax.ShapeDtypeStruct((_GPAD // 128, 128), jnp.float32),
                  jax.ShapeDtypeStruct((_GPAD // 128, 128), jnp.int32)),
        compiler_params=pltpu.CompilerParams(needs_layout_passes=False),
        scratch_types=[
            pltpu.VMEM((_GPT // 128, 128), jnp.int32),
            pltpu.VMEM((_GPT // 128, 128), jnp.int32),
            pltpu.VMEM((_GPT // 128, 128), jnp.int32),
            pltpu.VMEM((_GPT // 128, 128), jnp.float32),
            pltpu.VMEM((_GPT // 128, 128), jnp.int32),
            pltpu.SemaphoreType.DMA,
        ],
    )
    def k(idx_hbm, x_hbm, y_hbm, t_hbm, p_hbm,
          xo_hbm, yo_hbm, to_hbm, po_hbm,
          idx_v, xo_v, yo_v, to_v, po_v, sem):
        wid = lax.axis_index("s") * _NC + lax.axis_index("c")
        base = wid * _GPT
        row = pl.multiple_of(base // 128, 8)
        pltpu.sync_copy(idx_hbm.at[pl.ds(row, _GPT // 128)], idx_v)
        handles = []
        for r in range(_GPT // 128):
            handles.append(pltpu.async_copy(x_hbm.at[idx_v.at[r]], xo_v.at[r], sem))
            handles.append(pltpu.async_copy(y_hbm.at[idx_v.at[r]], yo_v.at[r], sem))
            handles.append(pltpu.async_copy(t_hbm.at[idx_v.at[r]], to_v.at[r], sem))
            handles.append(pltpu.async_copy(p_hbm.at[idx_v.at[r]], po_v.at[r], sem))
        for h in handles:
            h.wait()
        pltpu.sync_copy(xo_v, xo_hbm.at[pl.ds(row, _GPT // 128)])
        pltpu.sync_copy(yo_v, yo_hbm.at[pl.ds(row, _GPT // 128)])
        pltpu.sync_copy(to_v, to_hbm.at[pl.ds(row, _GPT // 128)])
        pltpu.sync_copy(po_v, po_hbm.at[pl.ds(row, _GPT // 128)])

    return k(idx_pad.reshape(_GPAD // 128, 128), x, y, t, p)


def _sc_gather_keys(table_pad, pix_pad):
    mesh = plsc.VectorSubcoreMesh(core_axis_name="c", subcore_axis_name="s", num_cores=_NC)

    @functools.partial(
        pl.kernel, mesh=mesh,
        out_type=jax.ShapeDtypeStruct((NEP,), jnp.int32),
        compiler_params=pltpu.CompilerParams(needs_layout_passes=False),
        scratch_types=[
            pltpu.VMEM((_TPAD,), jnp.int32),
            pltpu.VMEM((_CH,), jnp.int32),
            pltpu.VMEM((_CH,), jnp.int32),
        ],
    )
    def k(table_hbm, pix_hbm, keys_hbm, table_v, pix_v, key_v):
        wid = lax.axis_index("s") * _NC + lax.axis_index("c")
        base = wid * _EPT
        pltpu.sync_copy(table_hbm, table_v)
        for sub in range(_NSUB):
            off = base + sub * _CH
            pltpu.sync_copy(pix_hbm.at[pl.ds(off, _CH)], pix_v)

            def body(j, _):
                pv = pix_v[pl.ds(j * 16, 16)]
                key_v[pl.ds(j * 16, 16)] = plsc.load_gather(table_v, [pv])
                return 0

            lax.fori_loop(0, _CH // 16, body, 0)
            pltpu.sync_copy(key_v, keys_hbm.at[pl.ds(off, _CH)])

    return k(table_pad, pix_pad)


def _sc_scatter_body(keys_hbm, pos_hbm, kout_hbm, iout_hbm,
                     key_v, pos_v, idx_v, sent_v, kbuf_s, ibuf_s, sem):
    cid = lax.axis_index("c")
    sid = lax.axis_index("s")
    wid = sid * _NC + cid
    base = wid * _EPT
    iota16 = lax.broadcasted_iota(jnp.int32, (16,), 0)

    def initb(j, _):
        sent_v[pl.ds(j * 16, 16)] = jnp.full((16,), SENT_KEY, jnp.int32)
        return 0
    lax.fori_loop(0, (NBUF // _NS) // 16, initb, 0)
    pltpu.sync_copy(sent_v, kbuf_s.at[pl.ds(sid * (NBUF // _NS), NBUF // _NS)])
    pltpu.sync_copy(sent_v, ibuf_s.at[pl.ds(sid * (NBUF // _NS), NBUF // _NS)])
    plsc.subcore_barrier()

    for sub in range(_NSUB):
        off = base + sub * _CH
        row = wid * (_EPT // 128) + sub * (_CH // 128)
        pltpu.sync_copy(keys_hbm.at[pl.ds(row, _CH // 128)], key_v)
        pltpu.sync_copy(pos_hbm.at[pl.ds(row, _CH // 128)], pos_v)

        def fill(j, _):
            idx_v[j, pl.ds(0, 16)] = off + j * 128 + iota16
            idx_v[j, pl.ds(16, 16)] = off + j * 128 + 16 + iota16
            idx_v[j, pl.ds(32, 16)] = off + j * 128 + 32 + iota16
            idx_v[j, pl.ds(48, 16)] = off + j * 128 + 48 + iota16
            idx_v[j, pl.ds(64, 16)] = off + j * 128 + 64 + iota16
            idx_v[j, pl.ds(80, 16)] = off + j * 128 + 80 + iota16
            idx_v[j, pl.ds(96, 16)] = off + j * 128 + 96 + iota16
            idx_v[j, pl.ds(112, 16)] = off + j * 128 + 112 + iota16
            return 0
        lax.fori_loop(0, _CH // 128, fill, 0)

        handles = []
        for r in range(_CH // 128):
            handles.append(pltpu.async_copy(key_v.at[r], kbuf_s.at[pos_v.at[r]], sem))
            handles.append(pltpu.async_copy(idx_v.at[r], ibuf_s.at[pos_v.at[r]], sem))
        for h in handles:
            h.wait()

    plsc.subcore_barrier()

    @pl.when(sid == 0)
    def _out():
        pltpu.sync_copy(kbuf_s, kout_hbm.at[cid])
        pltpu.sync_copy(ibuf_s, iout_hbm.at[cid])


def _sc_scatter(keys_p, pos_p):
    mesh = plsc.VectorSubcoreMesh(core_axis_name="c", subcore_axis_name="s", num_cores=_NC)
    key2d = keys_p.reshape(NEP // 128, 128)
    pos2d = pos_p.reshape(NEP // 128, 128)

    @functools.partial(
        pl.kernel, mesh=mesh,
        out_type=(jax.ShapeDtypeStruct((_NC, NBUF), jnp.int32),
                  jax.ShapeDtypeStruct((_NC, NBUF), jnp.int32)),
        compiler_params=pltpu.CompilerParams(needs_layout_passes=False),
        scratch_types=[
            pltpu.VMEM((_CH // 128, 128), jnp.int32),
            pltpu.VMEM((_CH // 128, 128), jnp.int32),
            pltpu.VMEM((_CH // 128, 128), jnp.int32),
            pltpu.VMEM((NBUF // _NS,), jnp.int32),
            pltpu.VMEM_SHARED((NBUF,), jnp.int32),
            pltpu.VMEM_SHARED((NBUF,), jnp.int32),
            pltpu.SemaphoreType.DMA,
        ],
    )
    def k(keys_hbm, pos_hbm, kout_hbm, iout_hbm,
          key_v, pos_v, idx_v, sent_v, kbuf_s, ibuf_s, sem):
        _sc_scatter_body(keys_hbm, pos_hbm, kout_hbm, iout_hbm,
                         key_v, pos_v, idx_v, sent_v, kbuf_s, ibuf_s, sem)

    return k(key2d, pos2d)


def _roll1d(a, shift):
    if shift > 0:
        return jnp.concatenate([a[-shift:], a[:-shift]])
    s = -shift
    return jnp.concatenate([a[s:], a[:s]])


def _sort_body(keys, idxs, sc_out, idx_out):
    n = NBUF
    i1 = lax.broadcasted_iota(jnp.int32, (n,), 0)
    k2 = keys[...]
    i2 = idxs[...]
    use1 = k2[0] == SENT_KEY
    kraw = jnp.where(use1, k2[1], k2[0])
    iraw = jnp.where(use1, i2[1], i2[0])
    kk = jnp.where(i1 >= K, SENT_KEY, kraw)
    ii = jnp.where(i1 >= K, SENT_IDX, iraw)
    for ksz_log in range(1, 15):
        ksz = 1 << ksz_log
        j = ksz >> 1
        while j >= 1:
            lower = (i1 & j) == 0
            pk = jnp.where(lower, _roll1d(kk, -j), _roll1d(kk, j))
            pi = jnp.where(lower, _roll1d(ii, -j), _roll1d(ii, j))
            desc = (i1 & ksz) == 0
            pbetter = (pk > kk) | ((pk == kk) & (pi < ii))
            want_first = desc == lower
            take = want_first == pbetter
            kk = jnp.where(take, pk, kk)
            ii = jnp.where(take, pi, ii)
            j >>= 1
    neg = kk >= 0
    b = jnp.where(neg, kk, jnp.bitwise_not(jnp.bitwise_xor(kk, jnp.int32(-0x80000000))))
    sc_out[...] = lax.bitcast_convert_type(b, jnp.float32)
    idx_out[...] = ii


def _sort_topk(keybuf2, idxbuf2):
    return pl.pallas_call(
        _sort_body,
        out_shape=(jax.ShapeDtypeStruct((NBUF,), jnp.float32),
                   jax.ShapeDtypeStruct((NBUF,), jnp.int32)),
    )(keybuf2, idxbuf2)


def kernel(x, y, t, p, rgb, params):
    x = x.astype(jnp.int32)
    y = y.astype(jnp.int32)
    p = p.astype(jnp.int32)
    zpad = jnp.zeros((NEP - NE,), jnp.int32)
    xp = jnp.concatenate([x, zpad])
    yp = jnp.concatenate([y, zpad])
    tp = jnp.concatenate([t, zpad.astype(jnp.float32)])
    pp = jnp.concatenate([p, zpad])

    voxel = _sc_voxelize(xp, yp, tp, pp)
    rgb_f = _img_encoder(rgb, params)
    sm = _scorer(voxel, rgb_f, params)

    _s_tab, k_tab = _score_table_keys(sm)
    table_pad = jnp.full((_TPAD,), SENT_KEY, jnp.int32).at[:NPIX].set(k_tab)

    pix = y * SW + x
    pix_pad = jnp.full((NEP,), NPIX, jnp.int32).at[:NE].set(pix)

    keys_p = _sc_gather_keys(table_pad, pix_pad)

    theta, cgt = _find_theta(keys_p)
    pos_p = _positions(keys_p, theta, cgt).reshape(-1)

    keybuf2, idxbuf2 = _sc_scatter(keys_p, pos_p)

    top_scores, idx = _sort_topk(keybuf2, idxbuf2)
    top_scores = top_scores[:K]

    idx_pad = jnp.minimum(jnp.concatenate([idx, jnp.zeros((_GPAD - NBUF,), jnp.int32)]),
                          NE - 1)
    xo, yo, to, po = _sc_out_gather(idx_pad, x, y, t, p)
    xo = xo.reshape(-1)[:K]
    yo = yo.reshape(-1)[:K]
    to = to.reshape(-1)[:K]
    po = po.reshape(-1)[:K]

    return (xo, yo, to, po, top_scores, sm)


# speedup vs baseline: 15.0813x; 1.2341x over previous
import functools

import jax
import jax.numpy as jnp
import numpy as np
from jax import lax
from jax.experimental import pallas as pl
from jax.experimental.pallas import tpu as pltpu
from jax.experimental.pallas import tpu_sc as plsc

_NC, _NS = 2, 16
_NW = _NC * _NS
_EPT = (1 << 20) // _NW
_CH = 8192
_NSUB = _EPT // _CH
_TPAD = 68864

SH, SW, STRIDE, T, K, HID = 215, 320, 8, 8, 10000, 64
HG, WG = SH // STRIDE, SW // STRIDE
EPS = 1e-5
NPIX = SH * SW
NE = 1_000_000
NEP = 1 << 20
NBUF = 16384
TRASH = K
SENT_KEY = np.int32(-0x80000000)
SENT_IDX = np.int32(0x40000000)


def _conv(x, w, b=None, stride=1, pad=0):
    out = lax.conv_general_dilated(x, w, (stride, stride), [(pad, pad), (pad, pad)],
                                   dimension_numbers=('NCHW', 'OIHW', 'NCHW'))
    if b is not None:
        out = out + b[None, :, None, None]
    return out


def _bn(x, g, b):
    return x / jnp.sqrt(1.0 + EPS) * g[None, :, None, None] + b[None, :, None, None]


def _img_encoder(rgb, prm):
    h = jax.nn.relu(_bn(_conv(rgb, prm['stem0_w'], stride=2, pad=1), prm['bn0_g'], prm['bn0_b']))
    h = jax.nn.relu(_bn(_conv(h, prm['stem1_w'], stride=2, pad=1), prm['bn1_g'], prm['bn1_b']))
    h = jax.nn.relu(_bn(_conv(h, prm['stem2_w'], stride=2, pad=1), prm['bn2_g'], prm['bn2_b']))
    return _conv(h, prm['head_w'], prm['head_b'])


def _scorer(voxel, rgb_feat, prm):
    if rgb_feat.shape[-2:] != (HG, WG):
        rgb_feat = jax.image.resize(rgb_feat, (rgb_feat.shape[0], rgb_feat.shape[1], HG, WG),
                                    method='bilinear')
    e = _conv(voxel[None], prm['eproj_w'], prm['eproj_b'])
    r = _conv(rgb_feat, prm['rproj_w'], prm['rproj_b'])
    h = jnp.concatenate([e, r], axis=1)
    h = jax.nn.relu(_bn(_conv(h, prm['fuse0_w'], prm['fuse0_b'], pad=1), prm['fbn_g'], prm['fbn_b']))
    out = _conv(h, prm['fuse1_w'], prm['fuse1_b'])
    return out[0, 0]


def _pixel_score(x, y, sm):
    xn = x.astype(jnp.float32) / SW * 2.0 - 1.0
    yn = y.astype(jnp.float32) / SH * 2.0 - 1.0
    ix = ((xn + 1.0) * WG - 1.0) / 2.0
    iy = ((yn + 1.0) * HG - 1.0) / 2.0
    x0 = jnp.floor(ix); y0 = jnp.floor(iy)
    wx1 = ix - x0; wx0 = 1.0 - wx1
    wy1 = iy - y0; wy0 = 1.0 - wy1
    x0c = jnp.clip(x0, 0, WG - 1).astype(jnp.int32)
    x1c = jnp.clip(x0 + 1, 0, WG - 1).astype(jnp.int32)
    y0c = jnp.clip(y0, 0, HG - 1).astype(jnp.int32)
    y1c = jnp.clip(y0 + 1, 0, HG - 1).astype(jnp.int32)
    return (sm[y0c, x0c] * wy0 * wx0 + sm[y0c, x1c] * wy0 * wx1 +
            sm[y1c, x0c] * wy1 * wx0 + sm[y1c, x1c] * wy1 * wx1)


def _score_table_keys(sm):
    gx = jnp.tile(jnp.arange(SW, dtype=jnp.int32), SH)
    gy = jnp.repeat(jnp.arange(SH, dtype=jnp.int32), SW)
    s = _pixel_score(gx, gy, sm)
    b = lax.bitcast_convert_type(s, jnp.int32)
    kk = jnp.where(b < 0, jnp.bitwise_xor(jnp.bitwise_not(b), jnp.int32(-0x80000000)), b)
    return s, kk


def _theta_body(keys, theta_out, cgt_out):
    kk = keys[...]
    def step(b, c_u):
        trial_u = c_u | (jnp.uint32(1) << jnp.uint32(31 - b))
        trial_i = lax.bitcast_convert_type(trial_u ^ jnp.uint32(0x80000000), jnp.int32)
        cnt = jnp.sum((kk >= trial_i).astype(jnp.int32))
        return jnp.where(cnt >= K, trial_u, c_u)
    c_u = lax.fori_loop(0, 32, step, jnp.uint32(0))
    theta = lax.bitcast_convert_type(c_u ^ jnp.uint32(0x80000000), jnp.int32)
    theta_out[0, 0] = theta
    cgt_out[0, 0] = jnp.sum((kk > theta).astype(jnp.int32))


def _find_theta(keys_evt):
    return pl.pallas_call(
        _theta_body,
        out_shape=(jax.ShapeDtypeStruct((1, 1), jnp.int32),
                   jax.ShapeDtypeStruct((1, 1), jnp.int32)),
        out_specs=(pl.BlockSpec(memory_space=pltpu.SMEM),
                   pl.BlockSpec(memory_space=pltpu.SMEM)),
    )(keys_evt.reshape(8192, 128))


_SCAN_BLKS = 16
_SCAN_R = NEP // _SCAN_BLKS // 128


def _psum_rowmajor(x):
    c = x
    sh = 1
    while sh < 128:
        c = c + jnp.concatenate([jnp.zeros((c.shape[0], sh), jnp.int32), c[:, :-sh]], axis=1)
        sh *= 2
    rows = c[:, -1:]
    r = rows
    sh = 1
    while sh < x.shape[0]:
        r = r + jnp.concatenate([jnp.zeros((sh, 1), jnp.int32), r[:-sh, :]], axis=0)
        sh *= 2
    row_excl = r - rows
    return row_excl + (c - x)


def _pos_body(theta_ref, cgt_ref, keys, pos_out, acc):
    g = pl.program_id(0)

    @pl.when(g == 0)
    def _init():
        acc[0] = 0
        acc[1] = 0

    theta = theta_ref[0, 0]
    cgt = cgt_ref[0, 0]
    m = K - cgt
    kk = keys[...]
    f2 = (kk > theta).astype(jnp.int32)
    f1 = (kk == theta).astype(jnp.int32)
    ps2 = _psum_rowmajor(f2)
    ps1 = _psum_rowmajor(f1)
    base2 = acc[0]
    base1 = acc[1]
    eqr = base1 + ps1
    pos = jnp.where(f2 == 1, base2 + ps2,
                    jnp.where((f1 == 1) & (eqr < m), cgt + eqr, jnp.int32(TRASH)))
    pos_out[...] = pos
    acc[0] = base2 + jnp.sum(f2)
    acc[1] = base1 + jnp.sum(f1)


def _positions(keys_evt, theta, cgt):
    return pl.pallas_call(
        _pos_body,
        grid=(_SCAN_BLKS,),
        in_specs=[
            pl.BlockSpec(memory_space=pltpu.SMEM),
            pl.BlockSpec(memory_space=pltpu.SMEM),
            pl.BlockSpec((_SCAN_R, 128), lambda g: (g, 0)),
        ],
        out_specs=pl.BlockSpec((_SCAN_R, 128), lambda g: (g, 0)),
        out_shape=jax.ShapeDtypeStruct((_SCAN_BLKS * _SCAN_R, 128), jnp.int32),
        scratch_shapes=[pltpu.SMEM((2,), jnp.int32)],
    )(theta, cgt, keys_evt.reshape(_SCAN_BLKS * _SCAN_R, 128))


_NVOX = 2 * T * HG * WG
_VOXPAD = 16896


def _sc_voxelize(xp, yp, tp, pp):
    mesh = plsc.VectorSubcoreMesh(core_axis_name="c", subcore_axis_name="s", num_cores=_NC)

    @functools.partial(
        pl.kernel, mesh=mesh,
        out_type=jax.ShapeDtypeStruct((_NC, _VOXPAD), jnp.float32),
        compiler_params=pltpu.CompilerParams(needs_layout_passes=False),
        scratch_types=[
            pltpu.VMEM((_CH,), jnp.int32),
            pltpu.VMEM((_CH,), jnp.int32),
            pltpu.VMEM((_CH,), jnp.float32),
            pltpu.VMEM((_CH,), jnp.int32),
            pltpu.VMEM((_CH // 128, 128), jnp.int32),
            pltpu.VMEM((128,), jnp.float32),
            pltpu.VMEM((_VOXPAD // _NS,), jnp.float32),
            pltpu.VMEM_SHARED((_VOXPAD,), jnp.float32),
            pltpu.SemaphoreType.DMA,
        ],
    )
    def k(x_hbm, y_hbm, t_hbm, p_hbm, vout_hbm,
          x_v, y_v, t_v, p_v, flat_v, ones_v, zero_v, vox_s, sem):
        cid = lax.axis_index("c")
        sid = lax.axis_index("s")
        wid = sid * _NC + cid
        base = wid * _EPT
        iota16 = lax.broadcasted_iota(jnp.int32, (16,), 0)

        def zinit(j, _):
            zero_v[pl.ds(j * 16, 16)] = jnp.zeros((16,), jnp.float32)
            return 0
        lax.fori_loop(0, (_VOXPAD // _NS) // 16, zinit, 0)
        for c in range(8):
            ones_v[pl.ds(c * 16, 16)] = jnp.ones((16,), jnp.float32)
        pltpu.sync_copy(zero_v, vox_s.at[pl.ds(sid * (_VOXPAD // _NS), _VOXPAD // _NS)])
        plsc.subcore_barrier()

        for sub in range(_NSUB):
            off = base + sub * _CH
            pltpu.sync_copy(x_hbm.at[pl.ds(off, _CH)], x_v)
            pltpu.sync_copy(y_hbm.at[pl.ds(off, _CH)], y_v)
            pltpu.sync_copy(t_hbm.at[pl.ds(off, _CH)], t_v)
            pltpu.sync_copy(p_hbm.at[pl.ds(off, _CH)], p_v)

            def fill(r, _):
                for c in range(8):
                    sl = pl.ds(r * 128 + c * 16, 16)
                    xi = x_v[sl] >> 3
                    yi = jnp.minimum(y_v[sl] >> 3, HG - 1)
                    ti = jnp.clip((t_v[sl] * float(T)).astype(jnp.int32), 0, T - 1)
                    pib = jnp.where(p_v[sl] > 0, jnp.int32(T * HG * WG), jnp.int32(0))
                    flat = pib + ti * (HG * WG) + yi * WG + xi
                    gid = off + r * 128 + c * 16 + iota16
                    flat_v[r, pl.ds(c * 16, 16)] = jnp.where(gid < NE, flat, jnp.int32(_NVOX))
                return 0
            lax.fori_loop(0, _CH // 128, fill, 0)

            handles = []
            for r in range(_CH // 128):
                handles.append(pltpu.async_copy(ones_v, vox_s.at[flat_v.at[r]],
                                                sem, add=True))
            for h in handles:
                h.wait()

        plsc.subcore_barrier()

        @pl.when(sid == 0)
        def _out():
            pltpu.sync_copy(vox_s, vout_hbm.at[cid])

    v2 = k(xp, yp, tp, pp)
    return (v2[0, :_NVOX] + v2[1, :_NVOX]).reshape(2 * T, HG, WG)


_GPAD = 32768
_GPT = _GPAD // _NW


def _sc_out_gather(idx_pad, x, y, t, p):
    mesh = plsc.VectorSubcoreMesh(core_axis_name="c", subcore_axis_name="s", num_cores=_NC)

    @functools.partial(
        pl.kernel, mesh=mesh,
        out_type=(jax.ShapeDtypeStruct((_GPAD // 128, 128), jnp.int32),
                  jax.ShapeDtypeStruct((_GPAD // 128, 128), jnp.int32),
                  jax.ShapeDtypeStruct((_GPAD // 128, 128), jnp.float32),
                  jax.ShapeDtypeStruct((_GPAD // 128, 128), jnp.int32)),
        compiler_params=pltpu.CompilerParams(needs_layout_passes=False),
        scratch_types=[
            pltpu.VMEM((_GPT // 128, 128), jnp.int32),
            pltpu.VMEM((_GPT // 128, 128), jnp.int32),
            pltpu.VMEM((_GPT // 128, 128), jnp.int32),
            pltpu.VMEM((_GPT // 128, 128), jnp.float32),
            pltpu.VMEM((_GPT // 128, 128), jnp.int32),
            pltpu.SemaphoreType.DMA,
        ],
    )
    def k(idx_hbm, x_hbm, y_hbm, t_hbm, p_hbm,
          xo_hbm, yo_hbm, to_hbm, po_hbm,
          idx_v, xo_v, yo_v, to_v, po_v, sem):
        wid = lax.axis_index("s") * _NC + lax.axis_index("c")
        base = wid * _GPT
        row = pl.multiple_of(base // 128, 8)
        pltpu.sync_copy(idx_hbm.at[pl.ds(row, _GPT // 128)], idx_v)
        handles = []
        for r in range(_GPT // 128):
            handles.append(pltpu.async_copy(x_hbm.at[idx_v.at[r]], xo_v.at[r], sem))
            handles.append(pltpu.async_copy(y_hbm.at[idx_v.at[r]], yo_v.at[r], sem))
            handles.append(pltpu.async_copy(t_hbm.at[idx_v.at[r]], to_v.at[r], sem))
            handles.append(pltpu.async_copy(p_hbm.at[idx_v.at[r]], po_v.at[r], sem))
        for h in handles:
            h.wait()
        pltpu.sync_copy(xo_v, xo_hbm.at[pl.ds(row, _GPT // 128)])
        pltpu.sync_copy(yo_v, yo_hbm.at[pl.ds(row, _GPT // 128)])
        pltpu.sync_copy(to_v, to_hbm.at[pl.ds(row, _GPT // 128)])
        pltpu.sync_copy(po_v, po_hbm.at[pl.ds(row, _GPT // 128)])

    return k(idx_pad.reshape(_GPAD // 128, 128), x, y, t, p)


def _sc_gather_keys(table_pad, pix_pad):
    mesh = plsc.VectorSubcoreMesh(core_axis_name="c", subcore_axis_name="s", num_cores=_NC)

    @functools.partial(
        pl.kernel, mesh=mesh,
        out_type=jax.ShapeDtypeStruct((NEP,), jnp.int32),
        compiler_params=pltpu.CompilerParams(needs_layout_passes=False),
        scratch_types=[
            pltpu.VMEM((_TPAD,), jnp.int32),
            pltpu.VMEM((_CH,), jnp.int32),
            pltpu.VMEM((_CH,), jnp.int32),
        ],
    )
    def k(table_hbm, pix_hbm, keys_hbm, table_v, pix_v, key_v):
        wid = lax.axis_index("s") * _NC + lax.axis_index("c")
        base = wid * _EPT
        pltpu.sync_copy(table_hbm, table_v)
        for sub in range(_NSUB):
            off = base + sub * _CH
            pltpu.sync_copy(pix_hbm.at[pl.ds(off, _CH)], pix_v)

            def body(j, _):
                pv = pix_v[pl.ds(j * 16, 16)]
                key_v[pl.ds(j * 16, 16)] = plsc.load_gather(table_v, [pv])
                return 0

            lax.fori_loop(0, _CH // 16, body, 0)
            pltpu.sync_copy(key_v, keys_hbm.at[pl.ds(off, _CH)])

    return k(table_pad, pix_pad)


def _sc_scatter_body(keys_hbm, pos_hbm, kout_hbm, iout_hbm,
                     key_v, pos_v, kloc_v, iloc_v, iden_v, zero_v,
                     kbuf_s, ibuf_s, sem):
    cid = lax.axis_index("c")
    sid = lax.axis_index("s")
    wid = sid * _NC + cid
    base = wid * _EPT
    iota16 = lax.broadcasted_iota(jnp.int32, (16,), 0)

    def initz(j, _):
        zero_v[pl.ds(j * 16, 16)] = jnp.zeros((16,), jnp.int32)
        return 0
    lax.fori_loop(0, (NBUF // _NS) // 16, initz, 0)
    pltpu.sync_copy(zero_v, kbuf_s.at[pl.ds(sid * (NBUF // _NS), NBUF // _NS)])
    pltpu.sync_copy(zero_v, ibuf_s.at[pl.ds(sid * (NBUF // _NS), NBUF // _NS)])

    def initl(j, _):
        for c in range(8):
            sl = pl.ds(c * 16, 16)
            kloc_v[j, sl] = jnp.zeros((16,), jnp.int32)
            iloc_v[j, sl] = jnp.zeros((16,), jnp.int32)
            iden_v[j, sl] = j * 128 + c * 16 + iota16
        return 0
    lax.fori_loop(0, NBUF // 128, initl, 0)
    plsc.subcore_barrier()

    for sub in range(_NSUB):
        off = base + sub * _CH
        row = wid * (_EPT // 128) + sub * (_CH // 128)
        pltpu.sync_copy(keys_hbm.at[pl.ds(row, _CH // 128)], key_v)
        pltpu.sync_copy(pos_hbm.at[pl.ds(row, _CH // 128)], pos_v)

        def scat(j, _):
            for c in range(8):
                sl = pl.ds(c * 16, 16)
                pv = pos_v[j, sl]
                msk = pv != TRASH
                rows = lax.shift_right_logical(pv, 7)
                cols = pv & 127
                plsc.store_scatter(kloc_v, [rows, cols], key_v[j, sl], mask=msk)
                plsc.store_scatter(iloc_v, [rows, cols],
                                   off + j * 128 + c * 16 + iota16, mask=msk)
            return 0
        lax.fori_loop(0, _CH // 128, scat, 0)

    handles = []
    for r in range(NBUF // 128):
        handles.append(pltpu.async_copy(kloc_v.at[r], kbuf_s.at[iden_v.at[r]],
                                        sem, add=True))
        handles.append(pltpu.async_copy(iloc_v.at[r], ibuf_s.at[iden_v.at[r]],
                                        sem, add=True))
    for h in handles:
        h.wait()

    plsc.subcore_barrier()

    @pl.when(sid == 0)
    def _out():
        pltpu.sync_copy(kbuf_s, kout_hbm.at[cid])
        pltpu.sync_copy(ibuf_s, iout_hbm.at[cid])


def _sc_scatter(keys_p, pos_p):
    mesh = plsc.VectorSubcoreMesh(core_axis_name="c", subcore_axis_name="s", num_cores=_NC)
    key2d = keys_p.reshape(NEP // 128, 128)
    pos2d = pos_p.reshape(NEP // 128, 128)

    @functools.partial(
        pl.kernel, mesh=mesh,
        out_type=(jax.ShapeDtypeStruct((_NC, NBUF), jnp.int32),
                  jax.ShapeDtypeStruct((_NC, NBUF), jnp.int32)),
        compiler_params=pltpu.CompilerParams(needs_layout_passes=False),
        scratch_types=[
            pltpu.VMEM((_CH // 128, 128), jnp.int32),
            pltpu.VMEM((_CH // 128, 128), jnp.int32),
            pltpu.VMEM((NBUF // 128, 128), jnp.int32),
            pltpu.VMEM((NBUF // 128, 128), jnp.int32),
            pltpu.VMEM((NBUF // 128, 128), jnp.int32),
            pltpu.VMEM((NBUF // _NS,), jnp.int32),
            pltpu.VMEM_SHARED((NBUF,), jnp.int32),
            pltpu.VMEM_SHARED((NBUF,), jnp.int32),
            pltpu.SemaphoreType.DMA,
        ],
    )
    def k(keys_hbm, pos_hbm, kout_hbm, iout_hbm,
          key_v, pos_v, kloc_v, iloc_v, iden_v, zero_v, kbuf_s, ibuf_s, sem):
        _sc_scatter_body(keys_hbm, pos_hbm, kout_hbm, iout_hbm,
                         key_v, pos_v, kloc_v, iloc_v, iden_v, zero_v,
                         kbuf_s, ibuf_s, sem)

    return k(key2d, pos2d)


def _roll1d(a, shift):
    if shift > 0:
        return jnp.concatenate([a[-shift:], a[:-shift]])
    s = -shift
    return jnp.concatenate([a[s:], a[:s]])


def _sort_body(keys, idxs, sc_out, idx_out):
    n = NBUF
    i1 = lax.broadcasted_iota(jnp.int32, (n,), 0)
    k2 = keys[...]
    i2 = idxs[...]
    kk = jnp.where(i1 >= K, SENT_KEY, k2[0] + k2[1])
    ii = jnp.where(i1 >= K, SENT_IDX, i2[0] + i2[1])
    for ksz_log in range(1, 15):
        ksz = 1 << ksz_log
        j = ksz >> 1
        while j >= 1:
            lower = (i1 & j) == 0
            pk = jnp.where(lower, _roll1d(kk, -j), _roll1d(kk, j))
            pi = jnp.where(lower, _roll1d(ii, -j), _roll1d(ii, j))
            desc = (i1 & ksz) == 0
            pbetter = (pk > kk) | ((pk == kk) & (pi < ii))
            want_first = desc == lower
            take = want_first == pbetter
            kk = jnp.where(take, pk, kk)
            ii = jnp.where(take, pi, ii)
            j >>= 1
    neg = kk >= 0
    b = jnp.where(neg, kk, jnp.bitwise_not(jnp.bitwise_xor(kk, jnp.int32(-0x80000000))))
    sc_out[...] = lax.bitcast_convert_type(b, jnp.float32)
    idx_out[...] = ii


def _sort_topk(keybuf2, idxbuf2):
    return pl.pallas_call(
        _sort_body,
        out_shape=(jax.ShapeDtypeStruct((NBUF,), jnp.float32),
                   jax.ShapeDtypeStruct((NBUF,), jnp.int32)),
    )(keybuf2, idxbuf2)


def kernel(x, y, t, p, rgb, params):
    x = x.astype(jnp.int32)
    y = y.astype(jnp.int32)
    p = p.astype(jnp.int32)
    zpad = jnp.zeros((NEP - NE,), jnp.int32)
    xp = jnp.concatenate([x, zpad])
    yp = jnp.concatenate([y, zpad])
    tp = jnp.concatenate([t, zpad.astype(jnp.float32)])
    pp = jnp.concatenate([p, zpad])

    voxel = _sc_voxelize(xp, yp, tp, pp)
    rgb_f = _img_encoder(rgb, params)
    sm = _scorer(voxel, rgb_f, params)

    _s_tab, k_tab = _score_table_keys(sm)
    table_pad = jnp.full((_TPAD,), SENT_KEY, jnp.int32).at[:NPIX].set(k_tab)

    pix = y * SW + x
    pix_pad = jnp.full((NEP,), NPIX, jnp.int32).at[:NE].set(pix)

    keys_p = _sc_gather_keys(table_pad, pix_pad)

    theta, cgt = _find_theta(keys_p)
    pos_p = _positions(keys_p, theta, cgt).reshape(-1)

    keybuf2, idxbuf2 = _sc_scatter(keys_p, pos_p)

    top_scores, idx = _sort_topk(keybuf2, idxbuf2)
    top_scores = top_scores[:K]

    idx_pad = jnp.minimum(jnp.concatenate([idx, jnp.zeros((_GPAD - NBUF,), jnp.int32)]),
                          NE - 1)
    xo, yo, to, po = _sc_out_gather(idx_pad, x, y, t, p)
    xo = xo.reshape(-1)[:K]
    yo = yo.reshape(-1)[:K]
    to = to.reshape(-1)[:K]
    po = po.reshape(-1)[:K]

    return (xo, yo, to, po, top_scores, sm)
